# baseline (speedup 1.0000x reference)
import jax
import jax.numpy as jnp
from jax import lax
from jax.experimental import pallas as pl
from jax.experimental.pallas import tpu as pltpu
from jax.experimental.pallas import tpu_sc as plsc

LEVELS = 16
F = 2
BASE_RES = 16
SCALE = 1.5
TABLE = 524288
NPTS = 262144

NC, NS = 2, 16
NW = NC * NS
PW = NPTS // NW
C = 128
NCH = PW // C
NG = C // 16
NJ = LEVELS * 4 * F
RES = [int(BASE_RES * SCALE ** l) for l in range(LEVELS)]
H1, H2 = 73856093, 19349663
MASK = TABLE - 1

_KERNEL_KWARGS = dict(
    out_type=jax.ShapeDtypeStruct((NPTS * LEVELS * F,), jnp.float32),
    mesh=plsc.VectorSubcoreMesh(
        core_axis_name="c", subcore_axis_name="s",
        num_cores=NC, num_subcores=NS),
    scratch_types=[
        pltpu.VMEM((PW * 2,), jnp.float32),
        pltpu.VMEM((NJ, C), jnp.int32),
        pltpu.VMEM((LEVELS, 2, C), jnp.float32),
        pltpu.VMEM((NJ, C), jnp.float32),
        pltpu.VMEM((C * LEVELS * F,), jnp.float32),
        pltpu.SemaphoreType.DMA,
    ],
    compiler_params=pltpu.CompilerParams(
        needs_layout_passes=False, use_tc_tiling_on_sc=False),
)


def _encoder_body(xy_hbm, emb_hbm, out_hbm, xy_v, ibuf, wbuf, rbuf, obuf, gsem):
    i32 = jnp.int32
    _z, _o = i32(0), i32(1)
    wid = lax.axis_index("s") * i32(NC) + lax.axis_index("c")
    base = wid * i32(PW)
    pltpu.sync_copy(xy_hbm.at[pl.ds(base * i32(2), PW * 2)], xy_v)
    iota = lax.iota(jnp.int32, 16)
    iota2 = iota * i32(2)

    @pl.loop(_z, i32(NCH), step=_o)
    def _chunk(c):
        c0 = c * i32(C)

        @pl.loop(_z, i32(NG), step=_o)
        def _hash(g):
            p2 = iota2 + (c0 + g * i32(16)) * i32(2)
            ux = plsc.load_gather(xy_v, [p2])
            uy = plsc.load_gather(xy_v, [p2 + 1])
            ux = jnp.clip((ux + 1.0) * 0.5, 0.0, 1.0)
            uy = jnp.clip((uy + 1.0) * 0.5, 0.0, 1.0)
            for l in range(LEVELS):
                s = float(RES[l] - 1)
                px = ux * s
                py = uy * s
                x0 = px.astype(jnp.int32)
                y0 = py.astype(jnp.int32)
                wbuf[l, 0, pl.ds(g * 16, 16)] = px - x0.astype(jnp.float32)
                wbuf[l, 1, pl.ds(g * 16, 16)] = py - y0.astype(jnp.float32)
                x1 = jnp.minimum(x0 + 1, RES[l] - 1)
                y1 = jnp.minimum(y0 + 1, RES[l] - 1)
                a0 = x0 * H1
                a1 = x1 * H1
                b0 = y0 * H2
                b1 = y1 * H2
                lb2 = 2 * l * TABLE
                for k, (a, b) in enumerate(((a0, b0), (a1, b0),
                                            (a0, b1), (a1, b1))):
                    e2 = ((a ^ b) & MASK) * 2 + lb2
                    j = (l * 4 + k) * 2
                    ibuf[j, pl.ds(g * 16, 16)] = e2
                    ibuf[j + 1, pl.ds(g * 16, 16)] = e2 + 1

        @pl.loop(_z, i32(NJ), step=_o)
        def _fire(j):
            pltpu.async_copy(emb_hbm.at[ibuf.at[j]], rbuf.at[j], gsem)

        @pl.loop(_z, i32(NJ), step=_o)
        def _drain(j):
            pltpu.make_async_copy(emb_hbm.at[ibuf.at[j]],
                                  rbuf.at[j], gsem).wait()

        @pl.loop(_z, i32(NG), step=_o)
        def _combine(g):
            pids = iota + g * i32(16)
            p32 = pids * i32(LEVELS * F)
            for l in range(LEVELS):
                wx = wbuf[l, 0, pl.ds(g * 16, 16)]
                wy = wbuf[l, 1, pl.ds(g * 16, 16)]
                for f in range(F):
                    j0 = l * 8 + f
                    e00 = rbuf[j0 + 0, pl.ds(g * 16, 16)]
                    e10 = rbuf[j0 + 2, pl.ds(g * 16, 16)]
                    e01 = rbuf[j0 + 4, pl.ds(g * 16, 16)]
                    e11 = rbuf[j0 + 6, pl.ds(g * 16, 16)]
                    ex0 = e00 + (e10 - e00) * wx
                    ex1 = e01 + (e11 - e01) * wx
                    e = ex0 + (ex1 - ex0) * wy
                    plsc.store_scatter(obuf, [p32 + (2 * l + f)], e)

        pltpu.sync_copy(obuf, out_hbm.at[pl.ds((base + c0) * i32(LEVELS * F),
                                               C * LEVELS * F)])


_encoder = pl.kernel(_encoder_body, **_KERNEL_KWARGS)


def kernel(xy, emb):
    xyf = xy.reshape(NPTS * 2)
    embf = emb.reshape(LEVELS * TABLE * F)
    out = _encoder(xyf, embf)
    return out.reshape(NPTS, LEVELS * F)


# speedup vs baseline: 1.0221x; 1.0221x over previous
import jax
import jax.numpy as jnp
from jax import lax
from jax.experimental import pallas as pl
from jax.experimental.pallas import tpu as pltpu
from jax.experimental.pallas import tpu_sc as plsc

LEVELS = 16
F = 2
BASE_RES = 16
SCALE = 1.5
TABLE = 524288
NPTS = 262144

NC, NS = 2, 16
NW = NC * NS
PW = NPTS // NW
C = 128
NCH = PW // C
NG = C // 16
NJ = LEVELS * 4 * F
RES = [int(BASE_RES * SCALE ** l) for l in range(LEVELS)]
H1, H2 = 73856093, 19349663
MASK = TABLE - 1

_KERNEL_KWARGS = dict(
    out_type=jax.ShapeDtypeStruct((NPTS * LEVELS * F,), jnp.float32),
    mesh=plsc.VectorSubcoreMesh(
        core_axis_name="c", subcore_axis_name="s",
        num_cores=NC, num_subcores=NS),
    scratch_types=[
        pltpu.VMEM((PW * 2,), jnp.float32),
        pltpu.VMEM((NJ * C,), jnp.int32),
        pltpu.VMEM((NJ * C,), jnp.int32),
        pltpu.VMEM((LEVELS, 2, C), jnp.float32),
        pltpu.VMEM((LEVELS, 2, C), jnp.float32),
        pltpu.VMEM((NJ * C,), jnp.float32),
        pltpu.VMEM((NJ * C,), jnp.float32),
        pltpu.VMEM((C * LEVELS * F,), jnp.float32),
        pltpu.VMEM((C * LEVELS * F,), jnp.float32),
        pltpu.SemaphoreType.DMA,
        pltpu.SemaphoreType.DMA,
    ],
    compiler_params=pltpu.CompilerParams(
        needs_layout_passes=False, use_tc_tiling_on_sc=False),
)


def _encoder_body(xy_hbm, emb_hbm, out_hbm, xy_v,
                  ibufA, ibufB, wbufA, wbufB, rbufA, rbufB,
                  obufA, obufB, semA, semB):
    i32 = jnp.int32
    _z, _o = i32(0), i32(1)
    wid = lax.axis_index("s") * i32(NC) + lax.axis_index("c")
    base = wid * i32(PW)
    pltpu.sync_copy(xy_hbm.at[pl.ds(base * i32(2), PW * 2)], xy_v)
    iota = lax.iota(jnp.int32, 16)
    iota2 = iota * i32(2)

    def hash_stage(c0, ibuf, wbuf):
        @pl.loop(_z, i32(NG), step=_o)
        def _hash(g):
            g16 = g * i32(16)
            p2 = iota2 + (c0 + g16) * i32(2)
            ux = plsc.load_gather(xy_v, [p2])
            uy = plsc.load_gather(xy_v, [p2 + 1])
            ux = jnp.clip((ux + 1.0) * 0.5, 0.0, 1.0)
            uy = jnp.clip((uy + 1.0) * 0.5, 0.0, 1.0)
            for l in range(LEVELS):
                s = float(RES[l] - 1)
                px = ux * s
                py = uy * s
                x0 = px.astype(jnp.int32)
                y0 = py.astype(jnp.int32)
                wbuf[l, 0, pl.ds(g16, 16)] = px - x0.astype(jnp.float32)
                wbuf[l, 1, pl.ds(g16, 16)] = py - y0.astype(jnp.float32)
                x1 = jnp.minimum(x0 + 1, RES[l] - 1)
                y1 = jnp.minimum(y0 + 1, RES[l] - 1)
                a0 = x0 * H1
                a1 = x1 * H1
                b0 = y0 * H2
                b1 = y1 * H2
                lb2 = 2 * l * TABLE
                for k, (a, b) in enumerate(((a0, b0), (a1, b0),
                                            (a0, b1), (a1, b1))):
                    e2 = ((a ^ b) & MASK) * 2 + lb2
                    j = (l * 4 + k) * 2
                    ibuf[pl.ds(g16 + i32(j * C), 16)] = e2
                    ibuf[pl.ds(g16 + i32((j + 1) * C), 16)] = e2 + 1

    def fire(ibuf, rbuf, sem):
        pltpu.async_copy(emb_hbm.at[ibuf], rbuf, sem)

    def drain(ibuf, rbuf, sem):
        pltpu.make_async_copy(emb_hbm.at[ibuf], rbuf, sem).wait()

    def combine_store(c0, wbuf, rbuf, obuf):
        @pl.loop(_z, i32(NG), step=_o)
        def _combine(g):
            g16 = g * i32(16)
            p32 = (iota + g16) * i32(LEVELS * F)
            for l in range(LEVELS):
                wx = wbuf[l, 0, pl.ds(g16, 16)]
                wy = wbuf[l, 1, pl.ds(g16, 16)]
                for f in range(F):
                    j0 = (l * 8 + f) * C
                    e00 = rbuf[pl.ds(g16 + i32(j0 + 0 * C), 16)]
                    e10 = rbuf[pl.ds(g16 + i32(j0 + 2 * C), 16)]
                    e01 = rbuf[pl.ds(g16 + i32(j0 + 4 * C), 16)]
                    e11 = rbuf[pl.ds(g16 + i32(j0 + 6 * C), 16)]
                    ex0 = e00 + (e10 - e00) * wx
                    ex1 = e01 + (e11 - e01) * wx
                    e = ex0 + (ex1 - ex0) * wy
                    plsc.store_scatter(obuf, [p32 + (2 * l + f)], e)

        pltpu.sync_copy(obuf, out_hbm.at[pl.ds((base + c0) * i32(LEVELS * F),
                                               C * LEVELS * F)])

    hash_stage(_z, ibufA, wbufA)
    fire(ibufA, rbufA, semA)

    @pl.loop(_z, i32(NCH // 2), step=_o)
    def _pair(cc):
        ca0 = cc * i32(2 * C)
        cb0 = ca0 + i32(C)
        hash_stage(cb0, ibufB, wbufB)
        fire(ibufB, rbufB, semB)
        drain(ibufA, rbufA, semA)
        combine_store(ca0, wbufA, rbufA, obufA)

        @pl.when(cc < i32(NCH // 2 - 1))
        def _more():
            hash_stage(ca0 + i32(2 * C), ibufA, wbufA)
            fire(ibufA, rbufA, semA)

        drain(ibufB, rbufB, semB)
        combine_store(cb0, wbufB, rbufB, obufB)


_encoder = pl.kernel(_encoder_body, **_KERNEL_KWARGS)


def kernel(xy, emb):
    xyf = xy.reshape(NPTS * 2)
    embf = emb.reshape(LEVELS * TABLE * F)
    out = _encoder(xyf, embf)
    return out.reshape(NPTS, LEVELS * F)


# speedup vs baseline: 8.3906x; 8.2089x over previous
import jax
import jax.numpy as jnp
from jax import lax
from jax.experimental import pallas as pl
from jax.experimental.pallas import tpu as pltpu
from jax.experimental.pallas import tpu_sc as plsc

LEVELS = 16
F = 2
BASE_RES = 16
SCALE = 1.5
TABLE = 524288
NPTS = 262144

NC, NS = 2, 16
NW = NC * NS
PW = NPTS // NW
C = 128
NCH = PW // C
NG = C // 16
NJ = LEVELS * 4 * F
RES = [int(BASE_RES * SCALE ** l) for l in range(LEVELS)]
H1, H2 = 73856093, 19349663
MASK = TABLE - 1

_KERNEL_KWARGS = dict(
    out_type=jax.ShapeDtypeStruct((NPTS * LEVELS * F,), jnp.float32),
    mesh=plsc.VectorSubcoreMesh(
        core_axis_name="c", subcore_axis_name="s",
        num_cores=NC, num_subcores=NS),
    scratch_types=[
        pltpu.VMEM((PW * 2,), jnp.float32),
        pltpu.VMEM((NJ * C,), jnp.int32),
        pltpu.VMEM((NJ * C,), jnp.int32),
        pltpu.VMEM((LEVELS, 2, C), jnp.float32),
        pltpu.VMEM((LEVELS, 2, C), jnp.float32),
        pltpu.VMEM((NJ * C,), jnp.float32),
        pltpu.VMEM((NJ * C,), jnp.float32),
        pltpu.VMEM((C * LEVELS * F,), jnp.float32),
        pltpu.VMEM((C * LEVELS * F,), jnp.float32),
        pltpu.SemaphoreType.DMA,
        pltpu.SemaphoreType.DMA,
    ],
    compiler_params=pltpu.CompilerParams(
        needs_layout_passes=False, use_tc_tiling_on_sc=False),
)


def _encoder_body(xy_hbm, emb_hbm, out_hbm, xy_v,
                  ibufA, ibufB, wbufA, wbufB, rbufA, rbufB,
                  obufA, obufB, semA, semB):
    i32 = jnp.int32
    _z, _o = i32(0), i32(1)
    wid = lax.axis_index("s") * i32(NC) + lax.axis_index("c")
    base = wid * i32(PW)
    pltpu.sync_copy(xy_hbm.at[pl.ds(base * i32(2), PW * 2)], xy_v)
    iota = lax.iota(jnp.int32, 16)
    iota2 = iota * i32(2)

    def hash_stage(c0, ibuf, wbuf):
        @pl.loop(_z, i32(NG), step=_o)
        def _hash(g):
            g16 = g * i32(16)
            q = iota + (c0 + g16)
            qt = q & 127
            q0 = (q - qt) * 2 + qt
            ux = plsc.load_gather(xy_v, [q0])
            uy = plsc.load_gather(xy_v, [q0 + 128])
            ux = jnp.clip((ux + 1.0) * 0.5, 0.0, 1.0)
            uy = jnp.clip((uy + 1.0) * 0.5, 0.0, 1.0)
            for l in range(LEVELS):
                s = float(RES[l] - 1)
                px = ux * s
                py = uy * s
                x0 = px.astype(jnp.int32)
                y0 = py.astype(jnp.int32)
                wbuf[l, 0, pl.ds(g16, 16)] = px - x0.astype(jnp.float32)
                wbuf[l, 1, pl.ds(g16, 16)] = py - y0.astype(jnp.float32)
                x1 = jnp.minimum(x0 + 1, RES[l] - 1)
                y1 = jnp.minimum(y0 + 1, RES[l] - 1)
                a0 = x0 * H1
                a1 = x1 * H1
                b0 = y0 * H2
                b1 = y1 * H2
                lb = 2 * l * TABLE
                for k, (a, b) in enumerate(((a0, b0), (a1, b0),
                                            (a0, b1), (a1, b1))):
                    h = (a ^ b) & MASK
                    t = h & 127
                    e0 = (h - t) * 2 + t + lb
                    j = (l * 4 + k) * 2
                    ibuf[pl.ds(g16 + i32(j * C), 16)] = e0
                    ibuf[pl.ds(g16 + i32((j + 1) * C), 16)] = e0 + 128

    def fire(ibuf, rbuf, sem):
        pltpu.async_copy(emb_hbm.at[ibuf], rbuf, sem)

    def drain(ibuf, rbuf, sem):
        pltpu.make_async_copy(emb_hbm.at[ibuf], rbuf, sem).wait()

    def combine_store(c0, wbuf, rbuf, obuf):
        @pl.loop(_z, i32(NG), step=_o)
        def _combine(g):
            g16 = g * i32(16)
            p32 = (iota + g16) * i32(LEVELS * F)
            for l in range(LEVELS):
                wx = wbuf[l, 0, pl.ds(g16, 16)]
                wy = wbuf[l, 1, pl.ds(g16, 16)]
                for f in range(F):
                    j0 = (l * 8 + f) * C
                    e00 = rbuf[pl.ds(g16 + i32(j0 + 0 * C), 16)]
                    e10 = rbuf[pl.ds(g16 + i32(j0 + 2 * C), 16)]
                    e01 = rbuf[pl.ds(g16 + i32(j0 + 4 * C), 16)]
                    e11 = rbuf[pl.ds(g16 + i32(j0 + 6 * C), 16)]
                    ex0 = e00 + (e10 - e00) * wx
                    ex1 = e01 + (e11 - e01) * wx
                    e = ex0 + (ex1 - ex0) * wy
                    plsc.store_scatter(obuf, [p32 + (2 * l + f)], e)

        pltpu.sync_copy(obuf, out_hbm.at[pl.ds((base + c0) * i32(LEVELS * F),
                                               C * LEVELS * F)])

    hash_stage(_z, ibufA, wbufA)
    fire(ibufA, rbufA, semA)

    @pl.loop(_z, i32(NCH // 2), step=_o)
    def _pair(cc):
        ca0 = cc * i32(2 * C)
        cb0 = ca0 + i32(C)
        hash_stage(cb0, ibufB, wbufB)
        fire(ibufB, rbufB, semB)
        drain(ibufA, rbufA, semA)
        combine_store(ca0, wbufA, rbufA, obufA)

        @pl.when(cc < i32(NCH // 2 - 1))
        def _more():
            hash_stage(ca0 + i32(2 * C), ibufA, wbufA)
            fire(ibufA, rbufA, semA)

        drain(ibufB, rbufB, semB)
        combine_store(cb0, wbufB, rbufB, obufB)


_encoder = pl.kernel(_encoder_body, **_KERNEL_KWARGS)


def kernel(xy, emb):
    xyf = xy.reshape(NPTS // 128, 128, 2).transpose(0, 2, 1).reshape(NPTS * 2)
    embf = (emb.reshape(LEVELS, TABLE // 128, 128, F)
            .transpose(0, 1, 3, 2).reshape(LEVELS * TABLE * F))
    out = _encoder(xyf, embf)
    return out.reshape(NPTS, LEVELS * F)
